# baseline (speedup 1.0000x reference)
import functools

import jax
import jax.numpy as jnp
from jax import lax
from jax.experimental import pallas as pl
from jax.experimental.pallas import tpu as pltpu
from jax.experimental.pallas import tpu_sc as plsc

_K = 9
_N_TARGET = 4000
_N = 3456
_NW = 32
_RPW = _N // _NW
_NREP = 27
_PAD = 16
_INF = jnp.float32(jnp.inf)
_POISON = jnp.float32(1e4)


def _g16(v, idx):
    return jnp.take_along_axis(v, idx, axis=0, mode="promise_in_bounds")


def _sc_knn(x, y, z, sliced, start, rpw):
    mesh = plsc.VectorSubcoreMesh(core_axis_name="c", subcore_axis_name="s",
                                  num_cores=2, num_subcores=16)

    @functools.partial(
        pl.kernel,
        out_type=[
            jax.ShapeDtypeStruct((_NW, rpw * _PAD), jnp.int32),
            jax.ShapeDtypeStruct((_NW, rpw * _PAD), jnp.float32),
        ],
        mesh=mesh,
        scratch_types=[
            pltpu.VMEM((_N,), jnp.float32),
            pltpu.VMEM((_N,), jnp.float32),
            pltpu.VMEM((_N,), jnp.float32),
            pltpu.VMEM((_N,), jnp.float32),
            pltpu.VMEM((rpw * _PAD,), jnp.int32),
            pltpu.VMEM((rpw * _PAD,), jnp.float32),
        ],
        compiler_params=pltpu.CompilerParams(needs_layout_passes=False),
    )
    def knn(xh, yh, zh, slh, oidx, od2, xv, yv, zv, slv, iv, dv):
        wid = lax.axis_index("s") * 2 + lax.axis_index("c")
        pltpu.sync_copy(xh, xv)
        pltpu.sync_copy(yh, yv)
        pltpu.sync_copy(zh, zv)
        pltpu.sync_copy(slh, slv)
        base = start + wid * rpw
        iota = lax.broadcasted_iota(jnp.int32, (16,), 0)
        zero16i = jnp.zeros((16,), jnp.int32)
        inf16 = jnp.full((16,), _INF, jnp.float32)
        ninf16 = jnp.full((16,), -_INF, jnp.float32)
        k8 = jnp.full((16,), _K - 1, jnp.int32)
        kmask = iota > _K - 1
        shix = jnp.maximum(iota - 1, 0)

        def box_body(c, bc):
            (xla0, xla1, xha0, xha1, xlb0, xlb1, xhb0, xhb1,
             yla0, yla1, yha0, yha1, ylb0, ylb1, yhb0, yhb1,
             zla0, zla1, zha0, zha1, zlb0, zlb1, zhb0, zhb1) = bc
            cb0 = c * 128
            outs = []
            for arr in (xv, yv, zv):
                lo_a = inf16
                hi_a = ninf16
                lo_b = inf16
                hi_b = ninf16
                for v in range(8):
                    cb = cb0 + v * 16
                    vals = arr[pl.ds(cb, 16)]
                    sm = slv[pl.ds(cb, 16)] > jnp.float32(0.5)
                    lo_a = jnp.minimum(lo_a, jnp.where(sm, vals, inf16))
                    hi_a = jnp.maximum(hi_a, jnp.where(sm, vals, ninf16))
                    lo_b = jnp.minimum(lo_b, jnp.where(sm, inf16, vals))
                    hi_b = jnp.maximum(hi_b, jnp.where(sm, ninf16, vals))
                outs.append((jnp.min(lo_a), jnp.max(hi_a),
                             jnp.min(lo_b), jnp.max(hi_b)))
            m0 = iota == c
            m1 = iota == (c - 16)
            (xs, ys, zs) = outs
            return (
                jnp.where(m0, xs[0], xla0), jnp.where(m1, xs[0], xla1),
                jnp.where(m0, xs[1], xha0), jnp.where(m1, xs[1], xha1),
                jnp.where(m0, xs[2], xlb0), jnp.where(m1, xs[2], xlb1),
                jnp.where(m0, xs[3], xhb0), jnp.where(m1, xs[3], xhb1),
                jnp.where(m0, ys[0], yla0), jnp.where(m1, ys[0], yla1),
                jnp.where(m0, ys[1], yha0), jnp.where(m1, ys[1], yha1),
                jnp.where(m0, ys[2], ylb0), jnp.where(m1, ys[2], ylb1),
                jnp.where(m0, ys[3], yhb0), jnp.where(m1, ys[3], yhb1),
                jnp.where(m0, zs[0], zla0), jnp.where(m1, zs[0], zla1),
                jnp.where(m0, zs[1], zha0), jnp.where(m1, zs[1], zha1),
                jnp.where(m0, zs[2], zlb0), jnp.where(m1, zs[2], zlb1),
                jnp.where(m0, zs[3], zhb0), jnp.where(m1, zs[3], zhb1),
            )

        boxes = lax.fori_loop(0, _NREP, box_body, (inf16,) * 24)
        (xla0, xla1, xha0, xha1, xlb0, xlb1, xhb0, xhb1,
         yla0, yla1, yha0, yha1, ylb0, ylb1, yhb0, yhb1,
         zla0, zla1, zha0, zha1, zlb0, zlb1, zhb0, zhb1) = boxes

        def dist_vreg(cb, xi, yi, zi):
            dx = xi - xv[pl.ds(cb, 16)]
            acc = dx * dx
            dy = yi - yv[pl.ds(cb, 16)]
            acc = acc + dy * dy
            dz = zi - zv[pl.ds(cb, 16)]
            acc = acc + dz * dz
            return acc

        def merge_loop(acc, col, cb, carry):
            valb, idxb, t9 = carry
            idx9 = _g16(idxb, k8)
            mask0 = (acc < t9) | ((acc == t9) & (col < idx9))
            lane0 = plsc.all_reduce_ffs(mask0)[0]

            def wbody(st):
                mask, valb, idxb, t9, lane_s = st
                lane = zero16i + lane_s
                candv = _g16(acc, lane)
                candc = lane + cb
                pm = (valb < candv) | ((valb == candv) & (idxb < candc))
                pcnt = plsc.all_reduce_population_count(pm)
                shv = _g16(valb, shix)
                shi = _g16(idxb, shix)
                nv = jnp.where(iota < pcnt, valb,
                               jnp.where(iota == pcnt, candv, shv))
                ni = jnp.where(iota < pcnt, idxb,
                               jnp.where(iota == pcnt, candc, shi))
                valb2 = jnp.where(kmask, inf16, nv)
                idxb2 = ni
                t92 = _g16(valb2, k8)
                idx92 = _g16(idxb2, k8)
                mask2 = (mask & (iota != lane) &
                         ((acc < t92) | ((acc == t92) & (col < idx92))))
                lane2 = plsc.all_reduce_ffs(mask2)[0]
                return mask2, valb2, idxb2, t92, lane2

            def run(st):
                mask, valb, idxb, t9, lane_s = lax.while_loop(
                    lambda st: st[4] < 16, wbody, st)
                return mask, valb, idxb, t9, lane_s

            _, valb, idxb, t9, _ = lax.cond(
                lane0 < 16, run, lambda st: st,
                (mask0, valb, idxb, t9, lane0))
            return valb, idxb, t9

        def rescan(cb0, carry, xi, yi, zi, gi=None):
            for v in range(8):
                cb = cb0 + v * 16
                acc = dist_vreg(cb, xi, yi, zi)
                col = iota + cb
                if gi is not None:
                    acc = jnp.where(col == gi, acc + jnp.float32(1e9), acc)
                carry = merge_loop(acc, col, cb, carry)
            return carry

        def fast_chunk(cb0, carry, xi, yi, zi):
            t9 = carry[2]
            hit = dist_vreg(cb0, xi, yi, zi) <= t9
            for v in range(1, 8):
                hit = hit | (dist_vreg(cb0 + v * 16, xi, yi, zi) <= t9)
            anyhit = plsc.all_reduce_population_count(hit)[0]
            return lax.cond(anyhit > 0,
                            lambda c: rescan(cb0, c, xi, yi, zi),
                            lambda c: c, carry)

        def lb_half(lxa, hxa, lxb, hxb, lya, hya, lyb, hyb,
                    lza, hza, lzb, hzb, xi, yi, zi):
            zero = jnp.float32(0.0)
            dxa = jnp.maximum(jnp.maximum(lxa - xi, xi - hxa), zero)
            dya = jnp.maximum(jnp.maximum(lya - yi, yi - hya), zero)
            dza = jnp.maximum(jnp.maximum(lza - zi, zi - hza), zero)
            lba = dxa * dxa + dya * dya + dza * dza
            dxb = jnp.maximum(jnp.maximum(lxb - xi, xi - hxb), zero)
            dyb = jnp.maximum(jnp.maximum(lyb - yi, yi - hyb), zero)
            dzb = jnp.maximum(jnp.maximum(lzb - zi, zi - hzb), zero)
            lbb = dxb * dxb + dyb * dyb + dzb * dzb
            return jnp.minimum(lba, lbb) * jnp.float32(0.999)

        def row_body(r):
            gi = base + r
            gvec = (gi // 16) * 16
            glane = zero16i + (gi - gvec)
            xi = _g16(xv[pl.ds(gvec, 16)], glane)
            yi = _g16(yv[pl.ds(gvec, 16)], glane)
            zi = _g16(zv[pl.ds(gvec, 16)], glane)

            own = gi // 128
            carry = (inf16, zero16i, inf16)
            carry = rescan(own * 128, carry, xi, yi, zi, gi=gi)

            lb0 = lb_half(xla0, xha0, xlb0, xhb0, yla0, yha0, ylb0, yhb0,
                          zla0, zha0, zlb0, zhb0, xi, yi, zi)
            lb1 = lb_half(xla1, xha1, xlb1, xhb1, yla1, yha1, ylb1, yhb1,
                          zla1, zha1, zlb1, zhb1, xi, yi, zi)

            def chunk_body(c, carry):
                t9s = carry[2][0]
                cl = zero16i + c
                ch = zero16i + jnp.maximum(c - 16, 0)
                lbc = jnp.where(c < 16, _g16(lb0, cl), _g16(lb1, ch))[0]
                do = (c != own) & (lbc <= t9s)
                return lax.cond(do,
                                lambda cr: fast_chunk(c * 128, cr,
                                                      xi, yi, zi),
                                lambda cr: cr, carry)

            valb, idxb, _ = lax.fori_loop(0, _NREP, chunk_body, carry)

            dv[pl.ds(r * _PAD, 16)] = valb
            iv[pl.ds(r * _PAD, 16)] = idxb

        plsc.parallel_loop(0, rpw, 1, unroll=2)(row_body)
        pltpu.sync_copy(iv, oidx.at[wid])
        pltpu.sync_copy(dv, od2.at[wid])

    return knn(x, y, z, sliced)


_M = 2304
_TC_R = 288


def _tc_body(xr_ref, xc_ref, idx_ref, dst_ref):
    i = pl.program_id(0)
    xr = xr_ref[...]
    xc = xc_ref[...]
    d0 = xr[:, 0:1] - xc[0:1, :]
    acc = d0 * d0
    d1 = xr[:, 1:2] - xc[1:2, :]
    acc = acc + d1 * d1
    d2 = xr[:, 2:3] - xc[2:3, :]
    acc = acc + d2 * d2
    cols = jax.lax.broadcasted_iota(jnp.int32, (_TC_R, _N), 1)
    rows = jax.lax.broadcasted_iota(jnp.int32, (_TC_R, _N), 0) + i * _TC_R
    acc = jnp.where(cols == rows, acc + jnp.float32(1e9), acc)
    for k in range(_K):
        m = jnp.min(acc, axis=1, keepdims=True)
        hit = acc == m
        idx = jnp.min(jnp.where(hit, cols, _N), axis=1, keepdims=True)
        idx_ref[:, k:k + 1] = idx
        dst_ref[:, k:k + 1] = jnp.sqrt(jnp.maximum(m, jnp.float32(1e-12)))
        acc = jnp.where(cols == idx, jnp.float32(jnp.inf), acc)


def _tc_knn(x):
    xr = x[:_M]
    xc = x.T
    idx, dists = pl.pallas_call(
        _tc_body,
        grid=(_M // _TC_R,),
        in_specs=[
            pl.BlockSpec((_TC_R, 3), lambda i: (i, 0)),
            pl.BlockSpec((3, _N), lambda i: (0, 0)),
        ],
        out_specs=[
            pl.BlockSpec((_TC_R, _K), lambda i: (i, 0)),
            pl.BlockSpec((_TC_R, _K), lambda i: (i, 0)),
        ],
        out_shape=[
            jax.ShapeDtypeStruct((_M, _K), jnp.int32),
            jax.ShapeDtypeStruct((_M, _K), jnp.float32),
        ],
    )(xr, xc)
    return idx, dists


def kernel(positions, cell, numbers):
    frac = positions @ jnp.linalg.inv(cell)
    replicates = int((_N_TARGET / positions.shape[0]) ** (1.0 / 3.0))
    r = replicates
    ii, jj, kk = jnp.meshgrid(jnp.arange(r), jnp.arange(r), jnp.arange(r),
                              indexing='ij')
    offs = jnp.stack([ii, jj, kk], axis=-1).reshape(-1, 3).astype(frac.dtype)
    supercell = (frac[None, :, :] + offs[:, None, :]).reshape(-1, 3)

    scale = jnp.float32(0.05)
    eps = jax.random.normal(jax.random.key(42), supercell.shape,
                            supercell.dtype)
    supercell = supercell + scale * eps

    miller = jnp.array([1.0, 1.0, 0.0], dtype=jnp.float32)
    m = miller.astype(supercell.dtype)
    msum = jnp.sum(m)
    proj = supercell @ m
    thresh = replicates * msum / 2.0
    shift = jnp.where(proj > thresh, 1.0, 0.0).astype(supercell.dtype)
    supercell = supercell - shift[:, None] * (m / jnp.maximum(msum, 1.0)) * replicates

    supercell = supercell @ cell

    rpw = (_N - _M) // _NW
    oidx, od2 = _sc_knn(supercell[:, 0], supercell[:, 1], supercell[:, 2],
                        shift, _M, rpw)
    sc_src = oidx.reshape(_NW, rpw, _PAD)[:, :, :_K].reshape(_N - _M, _K)
    sc_d2 = od2.reshape(_NW, rpw, _PAD)[:, :, :_K].reshape(_N - _M, _K)
    sc_dists = jnp.sqrt(jnp.maximum(sc_d2, jnp.float32(1e-12)))
    tc_src, tc_dists = _tc_knn(supercell)
    src = jnp.concatenate([tc_src, sc_src], axis=0)
    dists = jnp.concatenate([tc_dists, sc_dists], axis=0)

    dst = jnp.broadcast_to(jnp.arange(_N)[:, None], (_N, _K))
    z = jnp.tile(numbers, r ** 3)
    return src, dst, dists, z, scale


# speedup vs baseline: 1.1293x; 1.1293x over previous
import functools

import jax
import jax.numpy as jnp
from jax import lax
from jax.experimental import pallas as pl
from jax.experimental.pallas import tpu as pltpu
from jax.experimental.pallas import tpu_sc as plsc

_K = 9
_N_TARGET = 4000
_N = 3456
_NW = 32
_RPW = _N // _NW
_NREP = 27
_PAD = 16
_INF = jnp.float32(jnp.inf)
_POISON = jnp.float32(1e4)


def _g16(v, idx):
    return jnp.take_along_axis(v, idx, axis=0, mode="promise_in_bounds")


def _sc_knn(x, y, z, sliced, start, rpw):
    mesh = plsc.VectorSubcoreMesh(core_axis_name="c", subcore_axis_name="s",
                                  num_cores=2, num_subcores=16)

    @functools.partial(
        pl.kernel,
        out_type=[
            jax.ShapeDtypeStruct((_NW, rpw * _PAD), jnp.int32),
            jax.ShapeDtypeStruct((_NW, rpw * _PAD), jnp.float32),
        ],
        mesh=mesh,
        scratch_types=[
            pltpu.VMEM((_N,), jnp.float32),
            pltpu.VMEM((_N,), jnp.float32),
            pltpu.VMEM((_N,), jnp.float32),
            pltpu.VMEM((_N,), jnp.float32),
            pltpu.VMEM((rpw * _PAD,), jnp.int32),
            pltpu.VMEM((rpw * _PAD,), jnp.float32),
        ],
        compiler_params=pltpu.CompilerParams(needs_layout_passes=False),
    )
    def knn(xh, yh, zh, slh, oidx, od2, xv, yv, zv, slv, iv, dv):
        wid = lax.axis_index("s") * 2 + lax.axis_index("c")
        pltpu.sync_copy(xh, xv)
        pltpu.sync_copy(yh, yv)
        pltpu.sync_copy(zh, zv)
        pltpu.sync_copy(slh, slv)
        base = start + wid * rpw
        iota = lax.broadcasted_iota(jnp.int32, (16,), 0)
        zero16i = jnp.zeros((16,), jnp.int32)
        inf16 = jnp.full((16,), _INF, jnp.float32)
        ninf16 = jnp.full((16,), -_INF, jnp.float32)
        k8 = jnp.full((16,), _K - 1, jnp.int32)
        kmask = iota > _K - 1
        shix = jnp.maximum(iota - 1, 0)

        def box_body(c, bc):
            (xla0, xla1, xha0, xha1, xlb0, xlb1, xhb0, xhb1,
             yla0, yla1, yha0, yha1, ylb0, ylb1, yhb0, yhb1,
             zla0, zla1, zha0, zha1, zlb0, zlb1, zhb0, zhb1) = bc
            cb0 = c * 128
            outs = []
            for arr in (xv, yv, zv):
                lo_a = inf16
                hi_a = ninf16
                lo_b = inf16
                hi_b = ninf16
                for v in range(8):
                    cb = cb0 + v * 16
                    vals = arr[pl.ds(cb, 16)]
                    sm = slv[pl.ds(cb, 16)] > jnp.float32(0.5)
                    lo_a = jnp.minimum(lo_a, jnp.where(sm, vals, inf16))
                    hi_a = jnp.maximum(hi_a, jnp.where(sm, vals, ninf16))
                    lo_b = jnp.minimum(lo_b, jnp.where(sm, inf16, vals))
                    hi_b = jnp.maximum(hi_b, jnp.where(sm, ninf16, vals))
                outs.append((jnp.min(lo_a), jnp.max(hi_a),
                             jnp.min(lo_b), jnp.max(hi_b)))
            m0 = iota == c
            m1 = iota == (c - 16)
            (xs, ys, zs) = outs
            return (
                jnp.where(m0, xs[0], xla0), jnp.where(m1, xs[0], xla1),
                jnp.where(m0, xs[1], xha0), jnp.where(m1, xs[1], xha1),
                jnp.where(m0, xs[2], xlb0), jnp.where(m1, xs[2], xlb1),
                jnp.where(m0, xs[3], xhb0), jnp.where(m1, xs[3], xhb1),
                jnp.where(m0, ys[0], yla0), jnp.where(m1, ys[0], yla1),
                jnp.where(m0, ys[1], yha0), jnp.where(m1, ys[1], yha1),
                jnp.where(m0, ys[2], ylb0), jnp.where(m1, ys[2], ylb1),
                jnp.where(m0, ys[3], yhb0), jnp.where(m1, ys[3], yhb1),
                jnp.where(m0, zs[0], zla0), jnp.where(m1, zs[0], zla1),
                jnp.where(m0, zs[1], zha0), jnp.where(m1, zs[1], zha1),
                jnp.where(m0, zs[2], zlb0), jnp.where(m1, zs[2], zlb1),
                jnp.where(m0, zs[3], zhb0), jnp.where(m1, zs[3], zhb1),
            )

        boxes = lax.fori_loop(0, _NREP, box_body, (inf16,) * 24)
        (xla0, xla1, xha0, xha1, xlb0, xlb1, xhb0, xhb1,
         yla0, yla1, yha0, yha1, ylb0, ylb1, yhb0, yhb1,
         zla0, zla1, zha0, zha1, zlb0, zlb1, zhb0, zhb1) = boxes

        def dist_vreg(cb, xi, yi, zi):
            dx = xi - xv[pl.ds(cb, 16)]
            acc = dx * dx
            dy = yi - yv[pl.ds(cb, 16)]
            acc = acc + dy * dy
            dz = zi - zv[pl.ds(cb, 16)]
            acc = acc + dz * dz
            return acc

        def merge_loop(acc, col, cb, carry):
            valb, idxb, t9 = carry
            idx9 = _g16(idxb, k8)
            mask0 = (acc < t9) | ((acc == t9) & (col < idx9))
            lane0 = plsc.all_reduce_ffs(mask0)[0]

            def wbody(st):
                mask, valb, idxb, t9, lane_s = st
                lane = zero16i + lane_s
                candv = _g16(acc, lane)
                candc = lane + cb
                pm = (valb < candv) | ((valb == candv) & (idxb < candc))
                pcnt = plsc.all_reduce_population_count(pm)
                shv = _g16(valb, shix)
                shi = _g16(idxb, shix)
                nv = jnp.where(iota < pcnt, valb,
                               jnp.where(iota == pcnt, candv, shv))
                ni = jnp.where(iota < pcnt, idxb,
                               jnp.where(iota == pcnt, candc, shi))
                valb2 = jnp.where(kmask, inf16, nv)
                idxb2 = ni
                t92 = _g16(valb2, k8)
                idx92 = _g16(idxb2, k8)
                mask2 = (mask & (iota != lane) &
                         ((acc < t92) | ((acc == t92) & (col < idx92))))
                lane2 = plsc.all_reduce_ffs(mask2)[0]
                return mask2, valb2, idxb2, t92, lane2

            def run(st):
                mask, valb, idxb, t9, lane_s = lax.while_loop(
                    lambda st: st[4] < 16, wbody, st)
                return mask, valb, idxb, t9, lane_s

            _, valb, idxb, t9, _ = lax.cond(
                lane0 < 16, run, lambda st: st,
                (mask0, valb, idxb, t9, lane0))
            return valb, idxb, t9

        def rescan(cb0, carry, xi, yi, zi, gi=None):
            for v in range(8):
                cb = cb0 + v * 16
                acc = dist_vreg(cb, xi, yi, zi)
                col = iota + cb
                if gi is not None:
                    acc = jnp.where(col == gi, acc + jnp.float32(1e9), acc)
                carry = merge_loop(acc, col, cb, carry)
            return carry

        def fast_chunk(cb0, carry, xi, yi, zi):
            t9 = carry[2]
            hit = dist_vreg(cb0, xi, yi, zi) <= t9
            for v in range(1, 8):
                hit = hit | (dist_vreg(cb0 + v * 16, xi, yi, zi) <= t9)
            anyhit = plsc.all_reduce_population_count(hit)[0]
            return lax.cond(anyhit > 0,
                            lambda c: rescan(cb0, c, xi, yi, zi),
                            lambda c: c, carry)

        def lb_half(lxa, hxa, lxb, hxb, lya, hya, lyb, hyb,
                    lza, hza, lzb, hzb, xi, yi, zi):
            zero = jnp.float32(0.0)
            dxa = jnp.maximum(jnp.maximum(lxa - xi, xi - hxa), zero)
            dya = jnp.maximum(jnp.maximum(lya - yi, yi - hya), zero)
            dza = jnp.maximum(jnp.maximum(lza - zi, zi - hza), zero)
            lba = dxa * dxa + dya * dya + dza * dza
            dxb = jnp.maximum(jnp.maximum(lxb - xi, xi - hxb), zero)
            dyb = jnp.maximum(jnp.maximum(lyb - yi, yi - hyb), zero)
            dzb = jnp.maximum(jnp.maximum(lzb - zi, zi - hzb), zero)
            lbb = dxb * dxb + dyb * dyb + dzb * dzb
            return jnp.minimum(lba, lbb) * jnp.float32(0.999)

        def row_body(r):
            gi = base + r
            gvec = (gi // 16) * 16
            glane = zero16i + (gi - gvec)
            xi = _g16(xv[pl.ds(gvec, 16)], glane)
            yi = _g16(yv[pl.ds(gvec, 16)], glane)
            zi = _g16(zv[pl.ds(gvec, 16)], glane)

            own = gi // 128
            carry = (inf16, zero16i, inf16)
            carry = rescan(own * 128, carry, xi, yi, zi, gi=gi)

            lb0 = lb_half(xla0, xha0, xlb0, xhb0, yla0, yha0, ylb0, yhb0,
                          zla0, zha0, zlb0, zhb0, xi, yi, zi)
            lb1 = lb_half(xla1, xha1, xlb1, xhb1, yla1, yha1, ylb1, yhb1,
                          zla1, zha1, zlb1, zhb1, xi, yi, zi)

            def chunk_body(c, carry):
                t9s = carry[2][0]
                cl = zero16i + c
                ch = zero16i + jnp.maximum(c - 16, 0)
                lbc = jnp.where(c < 16, _g16(lb0, cl), _g16(lb1, ch))[0]
                do = (c != own) & (lbc <= t9s)
                return lax.cond(do,
                                lambda cr: fast_chunk(c * 128, cr,
                                                      xi, yi, zi),
                                lambda cr: cr, carry)

            valb, idxb, _ = lax.fori_loop(0, _NREP, chunk_body, carry)

            dv[pl.ds(r * _PAD, 16)] = valb
            iv[pl.ds(r * _PAD, 16)] = idxb

        plsc.parallel_loop(0, rpw, 1, unroll=2)(row_body)
        pltpu.sync_copy(iv, oidx.at[wid])
        pltpu.sync_copy(dv, od2.at[wid])

    return knn(x, y, z, sliced)


_M = 2496
_TC_R = 312


def _tc_body(xr_ref, xc_ref, idx_ref, dst_ref):
    i = pl.program_id(0)
    xr = xr_ref[...]
    xc = xc_ref[...]
    d0 = xr[:, 0:1] - xc[0:1, :]
    acc = d0 * d0
    d1 = xr[:, 1:2] - xc[1:2, :]
    acc = acc + d1 * d1
    d2 = xr[:, 2:3] - xc[2:3, :]
    acc = acc + d2 * d2
    cols = jax.lax.broadcasted_iota(jnp.int32, (_TC_R, _N), 1)
    rows = jax.lax.broadcasted_iota(jnp.int32, (_TC_R, _N), 0) + i * _TC_R
    acc = jnp.where(cols == rows, acc + jnp.float32(1e9), acc)
    for k in range(_K):
        m = jnp.min(acc, axis=1, keepdims=True)
        hit = acc == m
        idx = jnp.min(jnp.where(hit, cols, _N), axis=1, keepdims=True)
        idx_ref[:, k:k + 1] = idx
        dst_ref[:, k:k + 1] = jnp.sqrt(jnp.maximum(m, jnp.float32(1e-12)))
        acc = jnp.where(cols == idx, jnp.float32(jnp.inf), acc)


def _tc_knn(x):
    xr = x[:_M]
    xc = x.T
    idx, dists = pl.pallas_call(
        _tc_body,
        grid=(_M // _TC_R,),
        in_specs=[
            pl.BlockSpec((_TC_R, 3), lambda i: (i, 0)),
            pl.BlockSpec((3, _N), lambda i: (0, 0)),
        ],
        out_specs=[
            pl.BlockSpec((_TC_R, _K), lambda i: (i, 0)),
            pl.BlockSpec((_TC_R, _K), lambda i: (i, 0)),
        ],
        out_shape=[
            jax.ShapeDtypeStruct((_M, _K), jnp.int32),
            jax.ShapeDtypeStruct((_M, _K), jnp.float32),
        ],
    )(xr, xc)
    return idx, dists


def kernel(positions, cell, numbers):
    frac = positions @ jnp.linalg.inv(cell)
    replicates = int((_N_TARGET / positions.shape[0]) ** (1.0 / 3.0))
    r = replicates
    ii, jj, kk = jnp.meshgrid(jnp.arange(r), jnp.arange(r), jnp.arange(r),
                              indexing='ij')
    offs = jnp.stack([ii, jj, kk], axis=-1).reshape(-1, 3).astype(frac.dtype)
    supercell = (frac[None, :, :] + offs[:, None, :]).reshape(-1, 3)

    scale = jnp.float32(0.05)
    eps = jax.random.normal(jax.random.key(42), supercell.shape,
                            supercell.dtype)
    supercell = supercell + scale * eps

    miller = jnp.array([1.0, 1.0, 0.0], dtype=jnp.float32)
    m = miller.astype(supercell.dtype)
    msum = jnp.sum(m)
    proj = supercell @ m
    thresh = replicates * msum / 2.0
    shift = jnp.where(proj > thresh, 1.0, 0.0).astype(supercell.dtype)
    supercell = supercell - shift[:, None] * (m / jnp.maximum(msum, 1.0)) * replicates

    supercell = supercell @ cell

    rpw = (_N - _M) // _NW
    oidx, od2 = _sc_knn(supercell[:, 0], supercell[:, 1], supercell[:, 2],
                        shift, _M, rpw)
    sc_src = oidx.reshape(_NW, rpw, _PAD)[:, :, :_K].reshape(_N - _M, _K)
    sc_d2 = od2.reshape(_NW, rpw, _PAD)[:, :, :_K].reshape(_N - _M, _K)
    sc_dists = jnp.sqrt(jnp.maximum(sc_d2, jnp.float32(1e-12)))
    tc_src, tc_dists = _tc_knn(supercell)
    src = jnp.concatenate([tc_src, sc_src], axis=0)
    dists = jnp.concatenate([tc_dists, sc_dists], axis=0)

    dst = jnp.broadcast_to(jnp.arange(_N)[:, None], (_N, _K))
    z = jnp.tile(numbers, r ** 3)
    return src, dst, dists, z, scale


# speedup vs baseline: 1.1610x; 1.0280x over previous
import functools

import jax
import jax.numpy as jnp
from jax import lax
from jax.experimental import pallas as pl
from jax.experimental.pallas import tpu as pltpu
from jax.experimental.pallas import tpu_sc as plsc

_K = 9
_N_TARGET = 4000
_N = 3456
_NW = 32
_RPW = _N // _NW
_NREP = 27
_PAD = 16
_INF = jnp.float32(jnp.inf)


def _g16(v, idx):
    return jnp.take_along_axis(v, idx, axis=0, mode="promise_in_bounds")


def _sc_knn(x, y, z, sliced, start, rpw):
    mesh = plsc.VectorSubcoreMesh(core_axis_name="c", subcore_axis_name="s",
                                  num_cores=2, num_subcores=16)

    @functools.partial(
        pl.kernel,
        out_type=[
            jax.ShapeDtypeStruct((_NW, rpw * _PAD), jnp.int32),
            jax.ShapeDtypeStruct((_NW, rpw * _PAD), jnp.float32),
        ],
        mesh=mesh,
        scratch_types=[
            pltpu.VMEM((_N,), jnp.float32),
            pltpu.VMEM((_N,), jnp.float32),
            pltpu.VMEM((_N,), jnp.float32),
            pltpu.VMEM((_N,), jnp.float32),
            pltpu.VMEM((rpw * _PAD,), jnp.int32),
            pltpu.VMEM((rpw * _PAD,), jnp.float32),
        ],
        compiler_params=pltpu.CompilerParams(needs_layout_passes=False),
    )
    def knn(xh, yh, zh, slh, oidx, od2, xv, yv, zv, slv, iv, dv):
        wid = lax.axis_index("s") * 2 + lax.axis_index("c")
        pltpu.sync_copy(xh, xv)
        pltpu.sync_copy(yh, yv)
        pltpu.sync_copy(zh, zv)
        pltpu.sync_copy(slh, slv)
        base = start + wid * rpw
        iota = lax.broadcasted_iota(jnp.int32, (16,), 0)
        zero16i = jnp.zeros((16,), jnp.int32)
        inf16 = jnp.full((16,), _INF, jnp.float32)
        ninf16 = jnp.full((16,), -_INF, jnp.float32)
        k8 = jnp.full((16,), _K - 1, jnp.int32)
        kmask = iota > _K - 1
        shix = jnp.maximum(iota - 1, 0)

        def box_body(c, bc):
            (xla0, xla1, xha0, xha1, xlb0, xlb1, xhb0, xhb1,
             yla0, yla1, yha0, yha1, ylb0, ylb1, yhb0, yhb1,
             zla0, zla1, zha0, zha1, zlb0, zlb1, zhb0, zhb1) = bc
            cb0 = c * 128
            outs = []
            for arr in (xv, yv, zv):
                lo_a = inf16
                hi_a = ninf16
                lo_b = inf16
                hi_b = ninf16
                for v in range(8):
                    cb = cb0 + v * 16
                    vals = arr[pl.ds(cb, 16)]
                    sm = slv[pl.ds(cb, 16)] > jnp.float32(0.5)
                    lo_a = jnp.minimum(lo_a, jnp.where(sm, vals, inf16))
                    hi_a = jnp.maximum(hi_a, jnp.where(sm, vals, ninf16))
                    lo_b = jnp.minimum(lo_b, jnp.where(sm, inf16, vals))
                    hi_b = jnp.maximum(hi_b, jnp.where(sm, ninf16, vals))
                outs.append((jnp.min(lo_a), jnp.max(hi_a),
                             jnp.min(lo_b), jnp.max(hi_b)))
            m0 = iota == c
            m1 = iota == (c - 16)
            (xs, ys, zs) = outs
            return (
                jnp.where(m0, xs[0], xla0), jnp.where(m1, xs[0], xla1),
                jnp.where(m0, xs[1], xha0), jnp.where(m1, xs[1], xha1),
                jnp.where(m0, xs[2], xlb0), jnp.where(m1, xs[2], xlb1),
                jnp.where(m0, xs[3], xhb0), jnp.where(m1, xs[3], xhb1),
                jnp.where(m0, ys[0], yla0), jnp.where(m1, ys[0], yla1),
                jnp.where(m0, ys[1], yha0), jnp.where(m1, ys[1], yha1),
                jnp.where(m0, ys[2], ylb0), jnp.where(m1, ys[2], ylb1),
                jnp.where(m0, ys[3], yhb0), jnp.where(m1, ys[3], yhb1),
                jnp.where(m0, zs[0], zla0), jnp.where(m1, zs[0], zla1),
                jnp.where(m0, zs[1], zha0), jnp.where(m1, zs[1], zha1),
                jnp.where(m0, zs[2], zlb0), jnp.where(m1, zs[2], zlb1),
                jnp.where(m0, zs[3], zhb0), jnp.where(m1, zs[3], zhb1),
            )

        boxes = lax.fori_loop(0, _NREP, box_body, (inf16,) * 24)
        (xla0, xla1, xha0, xha1, xlb0, xlb1, xhb0, xhb1,
         yla0, yla1, yha0, yha1, ylb0, ylb1, yhb0, yhb1,
         zla0, zla1, zha0, zha1, zlb0, zlb1, zhb0, zhb1) = boxes

        def dist_vreg(cb, xi, yi, zi):
            dx = xi - xv[pl.ds(cb, 16)]
            acc = dx * dx
            dy = yi - yv[pl.ds(cb, 16)]
            acc = acc + dy * dy
            dz = zi - zv[pl.ds(cb, 16)]
            acc = acc + dz * dz
            return acc

        def merge_loop(acc, col, cb, carry):
            valb, idxb, t9 = carry
            idx9 = _g16(idxb, k8)
            mask0 = (acc < t9) | ((acc == t9) & (col < idx9))
            lane0 = plsc.all_reduce_ffs(mask0)[0]

            def wbody(st):
                mask, valb, idxb, t9, lane_s = st
                lane = zero16i + lane_s
                candv = _g16(acc, lane)
                candc = lane + cb
                pm = (valb < candv) | ((valb == candv) & (idxb < candc))
                pcnt = plsc.all_reduce_population_count(pm)
                shv = _g16(valb, shix)
                shi = _g16(idxb, shix)
                nv = jnp.where(iota < pcnt, valb,
                               jnp.where(iota == pcnt, candv, shv))
                ni = jnp.where(iota < pcnt, idxb,
                               jnp.where(iota == pcnt, candc, shi))
                valb2 = jnp.where(kmask, inf16, nv)
                idxb2 = ni
                t92 = _g16(valb2, k8)
                idx92 = _g16(idxb2, k8)
                mask2 = (mask & (iota != lane) &
                         ((acc < t92) | ((acc == t92) & (col < idx92))))
                lane2 = plsc.all_reduce_ffs(mask2)[0]
                return mask2, valb2, idxb2, t92, lane2

            def run(st):
                mask, valb, idxb, t9, lane_s = lax.while_loop(
                    lambda st: st[4] < 16, wbody, st)
                return mask, valb, idxb, t9, lane_s

            _, valb, idxb, t9, _ = lax.cond(
                lane0 < 16, run, lambda st: st,
                (mask0, valb, idxb, t9, lane0))
            return valb, idxb, t9

        def rescan(cb0, carry, xi, yi, zi, gi=None):
            for v in range(8):
                cb = cb0 + v * 16
                acc = dist_vreg(cb, xi, yi, zi)
                col = iota + cb
                if gi is not None:
                    acc = jnp.where(col == gi, acc + jnp.float32(1e9), acc)
                carry = merge_loop(acc, col, cb, carry)
            return carry

        def fast_chunk(cb0, carry, xi, yi, zi):
            t9 = carry[2]
            hit = dist_vreg(cb0, xi, yi, zi) <= t9
            for v in range(1, 8):
                hit = hit | (dist_vreg(cb0 + v * 16, xi, yi, zi) <= t9)
            anyhit = plsc.all_reduce_population_count(hit)[0]
            return lax.cond(anyhit > 0,
                            lambda c: rescan(cb0, c, xi, yi, zi),
                            lambda c: c, carry)

        def lb_half(lxa, hxa, lxb, hxb, lya, hya, lyb, hyb,
                    lza, hza, lzb, hzb, xi, yi, zi):
            zero = jnp.float32(0.0)
            dxa = jnp.maximum(jnp.maximum(lxa - xi, xi - hxa), zero)
            dya = jnp.maximum(jnp.maximum(lya - yi, yi - hya), zero)
            dza = jnp.maximum(jnp.maximum(lza - zi, zi - hza), zero)
            lba = dxa * dxa + dya * dya + dza * dza
            dxb = jnp.maximum(jnp.maximum(lxb - xi, xi - hxb), zero)
            dyb = jnp.maximum(jnp.maximum(lyb - yi, yi - hyb), zero)
            dzb = jnp.maximum(jnp.maximum(lzb - zi, zi - hzb), zero)
            lbb = dxb * dxb + dyb * dyb + dzb * dzb
            return jnp.minimum(lba, lbb) * jnp.float32(0.999)

        def row_body(r):
            gi = base + r
            gvec = (gi // 16) * 16
            glane = zero16i + (gi - gvec)
            xi = _g16(xv[pl.ds(gvec, 16)], glane)
            yi = _g16(yv[pl.ds(gvec, 16)], glane)
            zi = _g16(zv[pl.ds(gvec, 16)], glane)

            own = gi // 128
            carry = (inf16, zero16i, inf16)
            carry = rescan(own * 128, carry, xi, yi, zi, gi=gi)

            lb0 = lb_half(xla0, xha0, xlb0, xhb0, yla0, yha0, ylb0, yhb0,
                          zla0, zha0, zlb0, zhb0, xi, yi, zi)
            lb1 = lb_half(xla1, xha1, xlb1, xhb1, yla1, yha1, ylb1, yhb1,
                          zla1, zha1, zlb1, zhb1, xi, yi, zi)

            def chunk_body(c, carry):
                t9s = carry[2][0]
                cl = zero16i + c
                ch = zero16i + jnp.maximum(c - 16, 0)
                lbc = jnp.where(c < 16, _g16(lb0, cl), _g16(lb1, ch))[0]
                do = (c != own) & (lbc <= t9s)
                return lax.cond(do,
                                lambda cr: fast_chunk(c * 128, cr,
                                                      xi, yi, zi),
                                lambda cr: cr, carry)

            valb, idxb, _ = lax.fori_loop(0, _NREP, chunk_body, carry)

            dv[pl.ds(r * _PAD, 16)] = valb
            iv[pl.ds(r * _PAD, 16)] = idxb

        plsc.parallel_loop(0, rpw, 1, unroll=2)(row_body)
        pltpu.sync_copy(iv, oidx.at[wid])
        pltpu.sync_copy(dv, od2.at[wid])

    return knn(x, y, z, sliced)


_M = 2560
_TC_R = 320


def _tc_body(xr_ref, xc_ref, idx_ref, dst_ref):
    i = pl.program_id(0)
    xr = xr_ref[...]
    xc = xc_ref[...]
    d0 = xr[:, 0:1] - xc[0:1, :]
    acc = d0 * d0
    d1 = xr[:, 1:2] - xc[1:2, :]
    acc = acc + d1 * d1
    d2 = xr[:, 2:3] - xc[2:3, :]
    acc = acc + d2 * d2
    cols = jax.lax.broadcasted_iota(jnp.int32, (_TC_R, _N), 1)
    rows = jax.lax.broadcasted_iota(jnp.int32, (_TC_R, _N), 0) + i * _TC_R
    acc = jnp.where(cols == rows, acc + jnp.float32(1e9), acc)
    for k in range(_K):
        m = jnp.min(acc, axis=1, keepdims=True)
        hit = acc == m
        idx = jnp.min(jnp.where(hit, cols, _N), axis=1, keepdims=True)
        idx_ref[:, k:k + 1] = idx
        dst_ref[:, k:k + 1] = jnp.sqrt(jnp.maximum(m, jnp.float32(1e-12)))
        acc = jnp.where(cols == idx, jnp.float32(jnp.inf), acc)


def _tc_knn(x):
    xr = x[:_M]
    xc = x.T
    idx, dists = pl.pallas_call(
        _tc_body,
        grid=(_M // _TC_R,),
        in_specs=[
            pl.BlockSpec((_TC_R, 3), lambda i: (i, 0)),
            pl.BlockSpec((3, _N), lambda i: (0, 0)),
        ],
        out_specs=[
            pl.BlockSpec((_TC_R, _K), lambda i: (i, 0)),
            pl.BlockSpec((_TC_R, _K), lambda i: (i, 0)),
        ],
        out_shape=[
            jax.ShapeDtypeStruct((_M, _K), jnp.int32),
            jax.ShapeDtypeStruct((_M, _K), jnp.float32),
        ],
    )(xr, xc)
    return idx, dists


def kernel(positions, cell, numbers):
    frac = positions @ jnp.linalg.inv(cell)
    replicates = int((_N_TARGET / positions.shape[0]) ** (1.0 / 3.0))
    r = replicates
    ii, jj, kk = jnp.meshgrid(jnp.arange(r), jnp.arange(r), jnp.arange(r),
                              indexing='ij')
    offs = jnp.stack([ii, jj, kk], axis=-1).reshape(-1, 3).astype(frac.dtype)
    supercell = (frac[None, :, :] + offs[:, None, :]).reshape(-1, 3)

    scale = jnp.float32(0.05)
    eps = jax.random.normal(jax.random.key(42), supercell.shape,
                            supercell.dtype)
    supercell = supercell + scale * eps

    miller = jnp.array([1.0, 1.0, 0.0], dtype=jnp.float32)
    m = miller.astype(supercell.dtype)
    msum = jnp.sum(m)
    proj = supercell @ m
    thresh = replicates * msum / 2.0
    shift = jnp.where(proj > thresh, 1.0, 0.0).astype(supercell.dtype)
    supercell = supercell - shift[:, None] * (m / jnp.maximum(msum, 1.0)) * replicates

    supercell = supercell @ cell

    rpw = (_N - _M) // _NW
    oidx, od2 = _sc_knn(supercell[:, 0], supercell[:, 1], supercell[:, 2],
                        shift, _M, rpw)
    sc_src = oidx.reshape(_NW, rpw, _PAD)[:, :, :_K].reshape(_N - _M, _K)
    sc_d2 = od2.reshape(_NW, rpw, _PAD)[:, :, :_K].reshape(_N - _M, _K)
    sc_dists = jnp.sqrt(jnp.maximum(sc_d2, jnp.float32(1e-12)))
    tc_src, tc_dists = _tc_knn(supercell)
    src = jnp.concatenate([tc_src, sc_src], axis=0)
    dists = jnp.concatenate([tc_dists, sc_dists], axis=0)

    dst = jnp.broadcast_to(jnp.arange(_N)[:, None], (_N, _K))
    z = jnp.tile(numbers, r ** 3)
    return src, dst, dists, z, scale
